# Initial kernel scaffold; baseline (speedup 1.0000x reference)
import jax
import jax.numpy as jnp
from jax.experimental import pallas as pl


def kernel(inputs, embeddings):
    raise NotImplementedError("write your pallas kernel here")


# speedup vs baseline: 1.4369x; 1.4369x over previous
import functools

import jax
import jax.numpy as jnp
from jax import lax
from jax.experimental import pallas as pl
from jax.experimental.pallas import tpu as pltpu
from jax.experimental.pallas import tpu_sc as plsc

ROWS = 16384
COLS = 26
D = 32
B = ROWS * COLS
NW = 32
BPW = B // NW
CHUNK = 128
NCHUNK = BPW // CHUNK

_mesh = plsc.VectorSubcoreMesh(core_axis_name="c", subcore_axis_name="s")


@functools.partial(
    pl.kernel,
    out_type=jax.ShapeDtypeStruct((B, D), jnp.float32),
    mesh=_mesh,
    compiler_params=pltpu.CompilerParams(use_tc_tiling_on_sc=False),
    scratch_types=[
        pltpu.VMEM((BPW,), jnp.int32),
        pltpu.VMEM((2, CHUNK, D), jnp.float32),
        pltpu.SemaphoreType.DMA,
        pltpu.SemaphoreType.DMA,
    ],
)
def _gather_kernel(idx_hbm, table_hbm, out_hbm, idx_v, rows_v, gsem, osem):
    wid = lax.axis_index("s") * 2 + lax.axis_index("c")
    base = wid * BPW

    pltpu.sync_copy(idx_hbm.at[pl.ds(base, BPW)], idx_v)

    def body(c, _):
        buf = lax.rem(c, 2)
        gather = pltpu.async_copy(
            table_hbm.at[idx_v.at[pl.ds(c * CHUNK, CHUNK)]], rows_v.at[buf], gsem
        )
        gather.wait()
        out = pltpu.async_copy(
            rows_v.at[buf], out_hbm.at[pl.ds(base + c * CHUNK, CHUNK)], osem
        )
        out.wait()
        return 0

    lax.fori_loop(0, NCHUNK, body, 0)


def kernel(inputs, embeddings):
    flat_idx = inputs.reshape(B)
    out = _gather_kernel(flat_idx, embeddings)
    return out.reshape(ROWS, COLS, D)


# speedup vs baseline: 1.5729x; 1.0947x over previous
import functools

import jax
import jax.numpy as jnp
from jax import lax
from jax.experimental import pallas as pl
from jax.experimental.pallas import tpu as pltpu
from jax.experimental.pallas import tpu_sc as plsc

ROWS = 16384
COLS = 26
D = 32
B = ROWS * COLS
NW = 32
BPW = B // NW
CHUNK = 128
NCHUNK = BPW // CHUNK
K = 4
NBUF = 2 * K

_mesh = plsc.VectorSubcoreMesh(core_axis_name="c", subcore_axis_name="s")


@functools.partial(
    pl.kernel,
    out_type=jax.ShapeDtypeStruct((B, D), jnp.float32),
    mesh=_mesh,
    compiler_params=pltpu.CompilerParams(use_tc_tiling_on_sc=False),
    scratch_types=[
        pltpu.VMEM((BPW,), jnp.int32),
        pltpu.VMEM((NBUF, CHUNK, D), jnp.float32),
        pltpu.SemaphoreType.DMA((NBUF,)),
        pltpu.SemaphoreType.DMA((NBUF,)),
    ],
)
def _gather_kernel(idx_hbm, table_hbm, out_hbm, idx_v, rows_v, gsem, osem):
    wid = lax.axis_index("s") * 2 + lax.axis_index("c")
    base = wid * BPW

    pltpu.sync_copy(idx_hbm.at[pl.ds(base, BPW)], idx_v)

    def start_gather(c, buf):
        pltpu.async_copy(
            table_hbm.at[idx_v.at[pl.ds(c * CHUNK, CHUNK)]],
            rows_v.at[buf],
            gsem.at[buf],
        )

    def wait_gather(buf):
        pltpu.make_async_copy(
            table_hbm.at[idx_v.at[pl.ds(0, CHUNK)]], rows_v.at[buf], gsem.at[buf]
        ).wait()

    def start_write(c, buf):
        pltpu.async_copy(
            rows_v.at[buf], out_hbm.at[pl.ds(base + c * CHUNK, CHUNK)], osem.at[buf]
        )

    def wait_write(buf):
        pltpu.make_async_copy(
            rows_v.at[buf], out_hbm.at[pl.ds(base, CHUNK)], osem.at[buf]
        ).wait()

    for c in range(K):
        start_gather(c, c)
    for c in range(K):
        wait_gather(c)
        start_write(c, c)
        start_gather(c + K, c + K)

    def body(c, _):
        buf = lax.rem(c, NBUF)
        wait_gather(buf)
        start_write(c, buf)
        buf2 = lax.rem(c + K, NBUF)
        wait_write(buf2)
        start_gather(c + K, buf2)
        return 0

    lax.fori_loop(K, NCHUNK - K, body, 0)

    for c in range(NCHUNK - K, NCHUNK):
        buf = c % NBUF
        wait_gather(buf)
        start_write(c, buf)
    for b in range(NBUF):
        wait_write(b)


def kernel(inputs, embeddings):
    flat_idx = inputs.reshape(B)
    out = _gather_kernel(flat_idx, embeddings)
    return out.reshape(ROWS, COLS, D)
